# baseline (speedup 1.0000x reference)
import functools

import jax
import jax.numpy as jnp
from jax import lax
from jax.experimental import pallas as pl
from jax.experimental.pallas import tpu as pltpu
from jax.experimental.pallas import tpu_sc as plsc


def _make_lookup(V, C, B):
    info = plsc.get_sparse_core_info()
    NC, NS, L = info.num_cores, info.num_subcores, info.num_lanes
    NW = NC * NS
    assert C == NW
    CHUNK = 4096
    n_chunks = B // CHUNK
    U = 8
    mesh = plsc.VectorSubcoreMesh(core_axis_name="c", subcore_axis_name="s")

    @functools.partial(
        pl.kernel,
        mesh=mesh,
        out_type=jax.ShapeDtypeStruct((C, B // 128, 128), jnp.float32),
        scratch_types=[
            pltpu.VMEM((V,), jnp.float32),
            pltpu.VMEM((B,), jnp.int32),
            pltpu.VMEM((CHUNK // 128, 128), jnp.float32),
            pltpu.VMEM((CHUNK // 128, 128), jnp.float32),
            pltpu.SemaphoreType.DMA,
            pltpu.SemaphoreType.DMA,
            pltpu.SemaphoreType.DMA,
        ],
        compiler_params=pltpu.CompilerParams(
            use_tc_tiling_on_sc=True, needs_layout_passes=False
        ),
    )
    def lookup_kernel(tableT_hbm, idx_hbm, outT_hbm, row_v, idx_v, val_a,
                      val_b, sem_row, sem_idx, sem_out):
        ch = lax.axis_index("s") * NC + lax.axis_index("c")
        row_cp = pltpu.async_copy(tableT_hbm.at[ch], row_v, sem_row)
        idx_cp = pltpu.async_copy(idx_hbm, idx_v, sem_idx)
        idx_cp.wait()
        row_cp.wait()

        bufs = (val_a, val_b)
        for k in range(n_chunks):
            buf = bufs[k % 2]
            rows = CHUNK // 128
            if k >= 2:
                pltpu.make_async_copy(
                    buf, outT_hbm.at[ch, pl.ds((k - 2) * rows, rows)], sem_out
                ).wait()

            def gather_body(j, _, k=k, buf=buf):
                base = k * CHUNK + j * (L * U)
                idx_vecs = [idx_v[pl.ds(base + u * L, L)] for u in range(U)]
                vals = [plsc.load_gather(row_v, [iv]) for iv in idx_vecs]
                for u in range(U):
                    buf[j, pl.ds(u * L, L)] = vals[u]
                return ()

            lax.fori_loop(0, CHUNK // (L * U), gather_body, (), unroll=1)
            pltpu.async_copy(
                buf, outT_hbm.at[ch, pl.ds(k * rows, rows)], sem_out
            )
        rows = CHUNK // 128
        for k in range(n_chunks - 2, n_chunks):
            pltpu.make_async_copy(
                bufs[k % 2], outT_hbm.at[ch, pl.ds(k * rows, rows)], sem_out
            ).wait()

    return lookup_kernel


def kernel(genre, table):
    B, = genre.shape
    V, C = table.shape
    outT = _make_lookup(V, C, B)(table.T, genre)
    return outT.reshape(C, B, 1, 1).transpose(1, 0, 2, 3)
